# baseline (speedup 1.0000x reference)
import functools

import jax
import jax.numpy as jnp
from jax import lax
from jax.experimental import pallas as pl
from jax.experimental.pallas import tpu as pltpu
from jax.experimental.pallas import tpu_sc as plsc

B, P, X, Y = 1024, 64, 32, 32
M = 4096
F = P * X * Y

NC, NS, L = 2, 16, 16
NW = NC * NS
BH = B // 2
MPAIR = M // (NW // 2)
NK = 8


def _policy_flatten_kernel(x_hbm, p_hbm, cx_hbm, cy_hbm, out_hbm,
                           pv, cxv, cyv, idx_v, g_v, s_v, gsem, osem):
  wid = lax.axis_index("s") * NC + lax.axis_index("c")
  pair = wid // 2
  b0 = pl.multiple_of((wid % 2) * BH, BH)

  lane = lax.iota(jnp.int32, L)

  def round_(r):
    m0 = pl.multiple_of(pair * MPAIR + r * 128, 128)

    pltpu.sync_copy(p_hbm.at[pl.ds(m0, 128)], pv)
    pltpu.sync_copy(cx_hbm.at[pl.ds(m0, 128)], cxv)
    pltpu.sync_copy(cy_hbm.at[pl.ds(m0, 128)], cyv)
    for j in range(128 // L):
      sl = pl.ds(j * L, L)
      idx_v[sl] = (pv[sl] * X + cxv[sl]) * Y + cyv[sl]

    def start_gather(k):
      chunk = idx_v[pl.ds(k * L, L)]
      for ml in range(L):
        row = lax.reduce_sum_p.bind(
            jnp.where(lane == ml, chunk, 0), axes=(0,))
        pltpu.async_copy(x_hbm.at[row, pl.ds(b0, BH)],
                         g_v.at[k % 2, ml], gsem)

    def drain_gather(k):
      pltpu.make_async_copy(x_hbm.at[pl.ds(0, L), pl.ds(0, BH)],
                            g_v.at[k % 2], gsem).wait()

    start_gather(0)
    for k in range(NK):
      if k + 1 < NK:
        start_gather(k + 1)
      drain_gather(k)
      if r == 1 and k == 0:
        pltpu.make_async_copy(s_v, out_hbm.at[pl.ds(0, BH), pl.ds(0, 128)],
                              osem).wait()
      buf = k % 2

      @functools.partial(plsc.parallel_loop, 0, L, unroll=2)
      def _(ml):
        col = jnp.full((L,), k * L, jnp.int32) + ml
        for j in range(BH // L):
          vals = g_v[buf, ml, pl.ds(j * L, L)]
          plsc.store_scatter(s_v, [lane + (j * L), col], vals)

    pltpu.async_copy(s_v, out_hbm.at[pl.ds(b0, BH), pl.ds(m0, 128)], osem)

  round_(0)
  round_(1)
  pltpu.make_async_copy(s_v, out_hbm.at[pl.ds(0, BH), pl.ds(0, 128)],
                        osem).wait()


@jax.jit
def kernel(x, piece_orientation_indices, center_placement_x,
           center_placement_y):
  xr = jnp.transpose(x, (1, 2, 3, 0)).reshape(F, B)
  run = pl.kernel(
      _policy_flatten_kernel,
      out_type=jax.ShapeDtypeStruct((B, M), jnp.float32),
      mesh=plsc.VectorSubcoreMesh(core_axis_name="c", subcore_axis_name="s"),
      scratch_types=[
          pltpu.VMEM((128,), jnp.int32),
          pltpu.VMEM((128,), jnp.int32),
          pltpu.VMEM((128,), jnp.int32),
          pltpu.VMEM((128,), jnp.int32),
          pltpu.VMEM((2, L, BH), jnp.float32),
          pltpu.VMEM((BH, 128), jnp.float32),
          pltpu.SemaphoreType.DMA,
          pltpu.SemaphoreType.DMA,
      ],
      compiler_params=pltpu.CompilerParams(needs_layout_passes=False),
  )
  return run(xr,
             piece_orientation_indices.astype(jnp.int32),
             center_placement_x.astype(jnp.int32),
             center_placement_y.astype(jnp.int32))


# speedup vs baseline: 1.2096x; 1.2096x over previous
import functools

import jax
import jax.numpy as jnp
from jax import lax
from jax.experimental import pallas as pl
from jax.experimental.pallas import tpu as pltpu
from jax.experimental.pallas import tpu_sc as plsc

B, P, X, Y = 1024, 64, 32, 32
M = 4096
F = P * X * Y

NC, NS, L = 2, 16, 16
NW = NC * NS
BH = B // 2
MPAIR = M // (NW // 2)
NK = 8


def _policy_flatten_kernel(x_hbm, p_hbm, cx_hbm, cy_hbm, out_hbm,
                           pv, cxv, cyv, idx_v, g_v, s_v, gsem, osem):
  wid = lax.axis_index("s") * NC + lax.axis_index("c")
  pair = wid // 2
  b0 = pl.multiple_of((wid % 2) * BH, BH)
  mp0 = pl.multiple_of(pair * MPAIR, 128)

  lane = lax.iota(jnp.int32, L)

  pltpu.sync_copy(p_hbm.at[pl.ds(mp0, MPAIR)], pv)
  pltpu.sync_copy(cx_hbm.at[pl.ds(mp0, MPAIR)], cxv)
  pltpu.sync_copy(cy_hbm.at[pl.ds(mp0, MPAIR)], cyv)
  for j in range(MPAIR // L):
    sl = pl.ds(j * L, L)
    idx_v[sl] = (pv[sl] * X + cxv[sl]) * Y + cyv[sl]

  def round_(r):
    m0 = pl.multiple_of(pair * MPAIR + r * 128, 128)

    def start_gather(k):
      src = x_hbm.at[:, pl.ds(b0, BH)].at[idx_v.at[pl.ds(r * 128 + k * L, L)]]
      return pltpu.async_copy(src, g_v.at[k % 2], gsem)

    def drain_gather(k):
      pltpu.make_async_copy(x_hbm.at[pl.ds(0, L), pl.ds(0, BH)],
                            g_v.at[k % 2], gsem).wait()

    start_gather(0)
    for k in range(NK):
      if k + 1 < NK:
        start_gather(k + 1)
      drain_gather(k)
      if r == 1 and k == 0:
        pltpu.make_async_copy(s_v, out_hbm.at[pl.ds(0, BH), pl.ds(0, 128)],
                              osem).wait()
      buf = k % 2

      @functools.partial(plsc.parallel_loop, 0, L, unroll=2)
      def _(ml):
        col = jnp.full((L,), k * L, jnp.int32) + ml
        for j in range(BH // L):
          vals = g_v[buf, ml, pl.ds(j * L, L)]
          plsc.store_scatter(s_v, [lane + (j * L), col], vals)

    pltpu.async_copy(s_v, out_hbm.at[pl.ds(b0, BH), pl.ds(m0, 128)], osem)

  round_(0)
  round_(1)
  pltpu.make_async_copy(s_v, out_hbm.at[pl.ds(0, BH), pl.ds(0, 128)],
                        osem).wait()


@jax.jit
def kernel(x, piece_orientation_indices, center_placement_x,
           center_placement_y):
  xr = jnp.transpose(x, (1, 2, 3, 0)).reshape(F, B)
  run = pl.kernel(
      _policy_flatten_kernel,
      out_type=jax.ShapeDtypeStruct((B, M), jnp.float32),
      mesh=plsc.VectorSubcoreMesh(core_axis_name="c", subcore_axis_name="s"),
      scratch_types=[
          pltpu.VMEM((MPAIR,), jnp.int32),
          pltpu.VMEM((MPAIR,), jnp.int32),
          pltpu.VMEM((MPAIR,), jnp.int32),
          pltpu.VMEM((MPAIR,), jnp.int32),
          pltpu.VMEM((2, L, BH), jnp.float32),
          pltpu.VMEM((BH, 128), jnp.float32),
          pltpu.SemaphoreType.DMA,
          pltpu.SemaphoreType.DMA,
      ],
      compiler_params=pltpu.CompilerParams(
          needs_layout_passes=False,
          disable_bounds_checks=True,
      ),
  )
  return run(xr,
             piece_orientation_indices.astype(jnp.int32),
             center_placement_x.astype(jnp.int32),
             center_placement_y.astype(jnp.int32))
